# baseline (speedup 1.0000x reference)
import functools

import jax
import jax.numpy as jnp
from jax import lax
from jax.experimental import pallas as pl
from jax.experimental.pallas import tpu as pltpu
from jax.experimental.pallas import tpu_sc as plsc

_NUM_CORES = 2
_NUM_SUBCORES = 16
_NUM_WORKERS = _NUM_CORES * _NUM_SUBCORES
_LANES = 16

_BATCH = 1024
_SEQ = 77
_HIDDEN = 768
_NVEC = _HIDDEN // _LANES
_ROWS = _BATCH * _SEQ
_BPW = _BATCH // _NUM_WORKERS
_CHUNK = 16
_NCHUNK = _BPW * _SEQ // _CHUNK
_DEPTH = 8
_DIST = 4


def _emb_body(ids_hbm, tok_hbm, pos_hbm, out_hbm, idx_v, pos_v, rows_v,
              *sems):
    sem_g = list(sems[:_DEPTH])
    sem_o = list(sems[_DEPTH:])
    wid = lax.axis_index("s") * _NUM_CORES + lax.axis_index("c")

    pltpu.sync_copy(ids_hbm.at[wid], idx_v)

    def parts(j):
        return lax.div(j, 2), lax.rem(j, 2)

    def gather_desc(j, slot):
        p, h = parts(j)
        return pltpu.make_async_copy(
            tok_hbm.at[idx_v.at[p, pl.ds(h * _CHUNK, _CHUNK)]],
            rows_v.at[pl.ds(slot * _CHUNK, _CHUNK)],
            sem_g[slot],
        )

    def pos_desc(j, slot):
        p, _ = parts(j)
        return pltpu.make_async_copy(pos_hbm.at[p], pos_v.at[slot], sem_g[slot])

    def out_desc(j, slot):
        p, h = parts(j)
        return pltpu.make_async_copy(
            rows_v.at[pl.ds(slot * _CHUNK, _CHUNK)],
            out_hbm.at[pl.ds(p * _BATCH + wid * _BPW + h * _CHUNK, _CHUNK)],
            sem_o[slot],
        )

    for j0 in range(_DIST):
        gather_desc(j0, j0).start()
        pos_desc(j0, j0).start()

    def outer(jo, carry):
        for b in range(_DEPTH):
            j = _DEPTH * jo + b
            nxt = (b + _DIST) % _DEPTH

            @pl.when(j < _NCHUNK)
            def _chunk():
                gather_desc(j, b).wait()
                pos_desc(j, b).wait()

                for cbase in range(0, _NVEC, _NVEC // 2):
                    pvecs = [pos_v[b, pl.ds((cbase + c) * _LANES, _LANES)]
                             for c in range(_NVEC // 2)]

                    def row_add(i, c, pvecs=pvecs, cbase=cbase):
                        r = b * _CHUNK + i
                        for c2 in range(_NVEC // 2):
                            sl = pl.ds((cbase + c2) * _LANES, _LANES)
                            rows_v[r, sl] = rows_v[r, sl] + pvecs[c2]
                        return c

                    lax.fori_loop(0, _CHUNK, row_add, 0)

                out_desc(j, b).start()

                @pl.when(j >= _DEPTH - _DIST)
                def _():
                    out_desc(j + _DIST - _DEPTH, nxt).wait()

                @pl.when(j + _DIST < _NCHUNK)
                def _():
                    gather_desc(j + _DIST, nxt).start()
                    pos_desc(j + _DIST, nxt).start()

        return carry

    lax.fori_loop(0, (_NCHUNK + _DEPTH - 1) // _DEPTH, outer, 0)

    for jt in range(_NCHUNK - (_DEPTH - _DIST), _NCHUNK):
        out_desc(jt, jt % _DEPTH).wait()


@jax.jit
def _emb_call(ids_pm, token_embedding, position_embedding):
    mesh = plsc.VectorSubcoreMesh(core_axis_name="c", subcore_axis_name="s")
    kern = functools.partial(
        pl.kernel,
        out_type=jax.ShapeDtypeStruct((_ROWS, _HIDDEN), jnp.float32),
        mesh=mesh,
        scratch_types=[
            pltpu.VMEM((_SEQ, _BPW), jnp.int32),
            pltpu.VMEM((_DEPTH, _HIDDEN), jnp.float32),
            pltpu.VMEM((_DEPTH * _CHUNK, _HIDDEN), jnp.float32),
        ] + [pltpu.SemaphoreType.DMA] * (2 * _DEPTH),
    )(_emb_body)
    return kern(ids_pm, token_embedding, position_embedding)


def kernel(input_ids, token_embedding, position_embedding):
    input_shape = input_ids.shape
    seq_len = input_shape[-1]
    flat_ids = input_ids.reshape(-1, seq_len).astype(jnp.int32)
    ids_pm = flat_ids.reshape(_NUM_WORKERS, _BPW, _SEQ).transpose(0, 2, 1)
    out = _emb_call(ids_pm, token_embedding, position_embedding)
    hidden = token_embedding.shape[-1]
    return out.reshape(seq_len, -1, hidden).transpose(1, 0, 2)


# speedup vs baseline: 1.0798x; 1.0798x over previous
import functools

import jax
import jax.numpy as jnp
from jax import lax
from jax.experimental import pallas as pl
from jax.experimental.pallas import tpu as pltpu
from jax.experimental.pallas import tpu_sc as plsc

_NUM_CORES = 2
_NUM_SUBCORES = 16
_NUM_WORKERS = _NUM_CORES * _NUM_SUBCORES
_LANES = 16

_BATCH = 1024
_SEQ = 77
_HIDDEN = 768
_NVEC = _HIDDEN // _LANES
_ROWS = _BATCH * _SEQ
_BPW = _BATCH // _NUM_WORKERS
_CHUNK = _BPW
_DEPTH = 4


def _emb_body(ids_hbm, tok_hbm, pos_hbm, out_hbm, idx_v, pos_v, rows_v,
              g0, g1, g2, g3, o0, o1, o2, o3):
    sem_g = [g0, g1, g2, g3]
    sem_o = [o0, o1, o2, o3]
    wid = lax.axis_index("s") * _NUM_CORES + lax.axis_index("c")

    pltpu.sync_copy(ids_hbm.at[wid], idx_v)

    def gather_desc(p, slot):
        return pltpu.make_async_copy(
            tok_hbm.at[idx_v.at[p]],
            rows_v.at[pl.ds(slot * _CHUNK, _CHUNK)],
            sem_g[slot],
        )

    def pos_desc(p, slot):
        return pltpu.make_async_copy(pos_hbm.at[p], pos_v.at[slot], sem_g[slot])

    def out_desc(p, slot, h):
        return pltpu.make_async_copy(
            rows_v.at[pl.ds(slot * _CHUNK + h * _LANES, _LANES)],
            out_hbm.at[pl.ds(p * _BATCH + wid * _BPW + h * _LANES, _LANES)],
            sem_o[slot],
        )

    for p in (0, 1):
        gather_desc(p, p).start()
        pos_desc(p, p).start()

    def outer(j4, carry):
        for b in range(_DEPTH):
            j = _DEPTH * j4 + b
            nxt = (b + 2) % _DEPTH

            @pl.when(j < _SEQ)
            def _chunk():
                gather_desc(j, b).wait()
                pos_desc(j, b).wait()

                for h in (0, 1):
                    for cbase in range(0, _NVEC, _NVEC // 2):
                        pvecs = [pos_v[b, pl.ds((cbase + c) * _LANES, _LANES)]
                                 for c in range(_NVEC // 2)]

                        def row_add(i, c, pvecs=pvecs, cbase=cbase, h=h):
                            r = b * _CHUNK + h * _LANES + i
                            for c2 in range(_NVEC // 2):
                                sl = pl.ds((cbase + c2) * _LANES, _LANES)
                                rows_v[r, sl] = rows_v[r, sl] + pvecs[c2]
                            return c

                        lax.fori_loop(0, _LANES, row_add, 0)

                    out_desc(j, b, h).start()

                @pl.when(j >= 2)
                def _():
                    out_desc(j - 2, nxt, 0).wait()
                    out_desc(j - 2, nxt, 1).wait()

                @pl.when(j + 2 < _SEQ)
                def _():
                    gather_desc(j + 2, nxt).start()
                    pos_desc(j + 2, nxt).start()

        return carry

    lax.fori_loop(0, (_SEQ + _DEPTH - 1) // _DEPTH, outer, 0)

    for pt in (_SEQ - 2, _SEQ - 1):
        for h in (0, 1):
            out_desc(pt, pt % _DEPTH, h).wait()


@jax.jit
def _emb_call(ids_pm, token_embedding, position_embedding):
    mesh = plsc.VectorSubcoreMesh(core_axis_name="c", subcore_axis_name="s")
    kern = functools.partial(
        pl.kernel,
        out_type=jax.ShapeDtypeStruct((_ROWS, _HIDDEN), jnp.float32),
        mesh=mesh,
        scratch_types=[
            pltpu.VMEM((_SEQ, _BPW), jnp.int32),
            pltpu.VMEM((_DEPTH, _HIDDEN), jnp.float32),
            pltpu.VMEM((_DEPTH * _CHUNK, _HIDDEN), jnp.float32),
            pltpu.SemaphoreType.DMA,
            pltpu.SemaphoreType.DMA,
            pltpu.SemaphoreType.DMA,
            pltpu.SemaphoreType.DMA,
            pltpu.SemaphoreType.DMA,
            pltpu.SemaphoreType.DMA,
            pltpu.SemaphoreType.DMA,
            pltpu.SemaphoreType.DMA,
        ],
    )(_emb_body)
    return kern(ids_pm, token_embedding, position_embedding)


def kernel(input_ids, token_embedding, position_embedding):
    input_shape = input_ids.shape
    seq_len = input_shape[-1]
    flat_ids = input_ids.reshape(-1, seq_len).astype(jnp.int32)
    ids_pm = flat_ids.reshape(_NUM_WORKERS, _BPW, _SEQ).transpose(0, 2, 1)
    out = _emb_call(ids_pm, token_embedding, position_embedding)
    hidden = token_embedding.shape[-1]
    return out.reshape(seq_len, -1, hidden).transpose(1, 0, 2)


# speedup vs baseline: 1.1112x; 1.0291x over previous
import functools

import jax
import jax.numpy as jnp
from jax import lax
from jax.experimental import pallas as pl
from jax.experimental.pallas import tpu as pltpu
from jax.experimental.pallas import tpu_sc as plsc

_NUM_CORES = 2
_NUM_SUBCORES = 16
_NUM_WORKERS = _NUM_CORES * _NUM_SUBCORES
_LANES = 16

_BATCH = 1024
_SEQ = 77
_HIDDEN = 768
_NVEC = _HIDDEN // _LANES
_ROWS = _BATCH * _SEQ
_BPW = _BATCH // _NUM_WORKERS
_CHUNK = _BPW
_DEPTH = 4


def _emb_body(ids_hbm, tok_hbm, pos_hbm, out_hbm, idx_v, pos_v, rows_v,
              g0, g1, g2, g3, o0, o1, o2, o3):
    sem_g = [g0, g1, g2, g3]
    sem_o = [o0, o1, o2, o3]
    wid = lax.axis_index("s") * _NUM_CORES + lax.axis_index("c")

    pltpu.sync_copy(ids_hbm.at[wid], idx_v)

    def gather_desc(p, slot):
        return pltpu.make_async_copy(
            tok_hbm.at[idx_v.at[p]],
            rows_v.at[pl.ds(slot * _CHUNK, _CHUNK)],
            sem_g[slot],
        )

    def pos_desc(p, slot):
        return pltpu.make_async_copy(pos_hbm.at[p], pos_v.at[slot], sem_g[slot])

    def out_desc(p, slot):
        return pltpu.make_async_copy(
            rows_v.at[pl.ds(slot * _CHUNK, _CHUNK)],
            out_hbm.at[pl.ds(p * _BATCH + wid * _BPW, _CHUNK)],
            sem_o[slot],
        )

    for p in (0, 1):
        gather_desc(p, p).start()
        pos_desc(p, p).start()

    def outer(j4, carry):
        for b in range(_DEPTH):
            j = _DEPTH * j4 + b
            nxt = (b + 2) % _DEPTH

            @pl.when(j < _SEQ)
            def _chunk():
                gather_desc(j, b).wait()
                pos_desc(j, b).wait()

                for cbase in range(0, _NVEC, _NVEC // 2):
                    pvecs = [pos_v[b, pl.ds((cbase + c) * _LANES, _LANES)]
                             for c in range(_NVEC // 2)]

                    @plsc.parallel_loop(0, _CHUNK, step=1, unroll=2)
                    def row_add(i, pvecs=pvecs, cbase=cbase):
                        r = b * _CHUNK + i
                        for c2 in range(_NVEC // 2):
                            sl = pl.ds((cbase + c2) * _LANES, _LANES)
                            rows_v[r, sl] = rows_v[r, sl] + pvecs[c2]

                out_desc(j, b).start()

                @pl.when(j >= 2)
                def _():
                    out_desc(j - 2, nxt).wait()

                @pl.when(j + 2 < _SEQ)
                def _():
                    gather_desc(j + 2, nxt).start()
                    pos_desc(j + 2, nxt).start()

        return carry

    lax.fori_loop(0, (_SEQ + _DEPTH - 1) // _DEPTH, outer, 0)

    out_desc(_SEQ - 2, (_SEQ - 2) % _DEPTH).wait()
    out_desc(_SEQ - 1, (_SEQ - 1) % _DEPTH).wait()


@jax.jit
def _emb_call(ids_pm, token_embedding, position_embedding):
    mesh = plsc.VectorSubcoreMesh(core_axis_name="c", subcore_axis_name="s")
    kern = functools.partial(
        pl.kernel,
        out_type=jax.ShapeDtypeStruct((_ROWS, _HIDDEN), jnp.float32),
        mesh=mesh,
        scratch_types=[
            pltpu.VMEM((_SEQ, _BPW), jnp.int32),
            pltpu.VMEM((_DEPTH, _HIDDEN), jnp.float32),
            pltpu.VMEM((_DEPTH * _CHUNK, _HIDDEN), jnp.float32),
            pltpu.SemaphoreType.DMA,
            pltpu.SemaphoreType.DMA,
            pltpu.SemaphoreType.DMA,
            pltpu.SemaphoreType.DMA,
            pltpu.SemaphoreType.DMA,
            pltpu.SemaphoreType.DMA,
            pltpu.SemaphoreType.DMA,
            pltpu.SemaphoreType.DMA,
        ],
    )(_emb_body)
    return kern(ids_pm, token_embedding, position_embedding)


def kernel(input_ids, token_embedding, position_embedding):
    input_shape = input_ids.shape
    seq_len = input_shape[-1]
    flat_ids = input_ids.reshape(-1, seq_len).astype(jnp.int32)
    ids_pm = flat_ids.reshape(_NUM_WORKERS, _BPW, _SEQ).transpose(0, 2, 1)
    out = _emb_call(ids_pm, token_embedding, position_embedding)
    hidden = token_embedding.shape[-1]
    return out.reshape(seq_len, -1, hidden).transpose(1, 0, 2)
